# baseline (speedup 1.0000x reference)
import functools

import jax
import jax.numpy as jnp
from jax.experimental import pallas as pl
from jax.experimental.pallas import tpu as pltpu

B = 4608
L = 256
K = 1024
RB = 64
KT = 128
NBR = B // RB
NK = K // KT


def _tree_sum_L(y):
    z = y[:, :128, :] + y[:, 128:, :]
    zr = z.reshape(z.shape[0], 16, 8, z.shape[-1])
    acc = zr[:, 0]
    for g in range(1, 16):
        acc = acc + zr[:, g]
    t = acc[:, :4, :] + acc[:, 4:, :]
    u = t[:, :2, :] + t[:, 2:, :]
    return u[:, 0, :] + u[:, 1, :]


def _dist_kernel(x_ref, e_ref, idx_ref, m1_ref, d2_ref):
    k = pl.program_id(1)
    xb = x_ref[...]
    et = e_ref[...]
    diff = xb[:, :, None] - et[None, :, :]
    d2t = _tree_sum_L(diff * diff)
    dt = jnp.sqrt(d2t)
    lmin = jnp.min(dt, axis=1)
    tied = dt == lmin[:, None]
    lanes = jax.lax.broadcasted_iota(jnp.int32, dt.shape, 1)
    lidx = jnp.min(jnp.where(tied, lanes, KT), axis=1)
    ld2 = jnp.min(jnp.where(tied, d2t, jnp.inf), axis=1)
    gidx = k * KT + lidx

    @pl.when(k == 0)
    def _():
        idx_ref[...] = gidx[None, None, :]
        m1_ref[...] = lmin[None, None, :]
        d2_ref[...] = ld2[None, None, :]

    @pl.when(k > 0)
    def _():
        m1 = m1_ref[...]
        better = lmin[None, None, :] < m1
        m1_ref[...] = jnp.where(better, lmin[None, None, :], m1)
        idx_ref[...] = jnp.where(better, gidx[None, None, :], idx_ref[...])
        d2_ref[...] = jnp.where(better, ld2[None, None, :], d2_ref[...])


def _compute_index_pallas(x2d, e2d):
    idx, m1, d2 = pl.pallas_call(
        _dist_kernel,
        grid=(NBR, NK),
        in_specs=[
            pl.BlockSpec((RB, L), lambda r, k: (r, 0)),
            pl.BlockSpec((L, KT), lambda r, k: (0, k)),
        ],
        out_specs=[
            pl.BlockSpec((1, 1, RB), lambda r, k: (r, 0, 0)),
            pl.BlockSpec((1, 1, RB), lambda r, k: (r, 0, 0)),
            pl.BlockSpec((1, 1, RB), lambda r, k: (r, 0, 0)),
        ],
        out_shape=[
            jax.ShapeDtypeStruct((NBR, 1, RB), jnp.int32),
            jax.ShapeDtypeStruct((NBR, 1, RB), jnp.float32),
            jax.ShapeDtypeStruct((NBR, 1, RB), jnp.float32),
        ],
        compiler_params=pltpu.CompilerParams(
            dimension_semantics=("parallel", "arbitrary"),
        ),
    )(x2d, e2d.T)
    return idx.reshape(B), d2.reshape(B)


def kernel(x0, embedding0):
    n_channels, n_classes, vec_len = embedding0.shape
    x2d = x0.reshape(B, L)
    e2d = embedding0.reshape(K, L)
    index, d2min = _compute_index_pallas(x2d, e2d)

    hist = jnp.bincount(index, length=n_classes).astype(jnp.float32)
    prob = hist / B
    safe_prob = jnp.where(hist > 0, prob, 1.0)
    entropy = -jnp.sum(jnp.where(hist > 0, prob * jnp.log(safe_prob), 0.0))
    output = jnp.take(e2d, index, axis=0)
    out0 = ((output - x2d) + x2d).reshape(x0.shape)
    out1 = d2min.reshape(x0.shape[0], x0.shape[1], x0.shape[2])
    return (out0, out1, out1, entropy)


# speedup vs baseline: 2.9676x; 2.9676x over previous
import functools

import jax
import jax.numpy as jnp
from jax import lax
from jax.experimental import pallas as pl
from jax.experimental.pallas import tpu as pltpu
from jax.experimental.pallas import tpu_sc as plsc

B = 4608
L = 256
K = 1024
C = 6

RB1 = 512
NBR1 = B // RB1

RB3 = 128
NBR3 = B // RB3


def _tree_sum(y):
    z = y[..., :128] + y[..., 128:]
    zr = z.reshape(z.shape[:-1] + (16, 8))
    acc = zr[..., 0, :]
    for g in range(1, 16):
        acc = acc + zr[..., g, :]
    t = acc[..., :4] + acc[..., 4:]
    u = t[..., :2] + t[..., 2:]
    return u[..., 0] + u[..., 1]


def _topc_kernel(x_ref, et_ref, cid_ref):
    xb = x_ref[...]
    et = et_ref[...]
    e2 = jnp.sum(et * et, axis=0, keepdims=True)
    s = jax.lax.dot_general(
        xb, et, (((1,), (0,)), ((), ())),
        precision=jax.lax.Precision.HIGHEST,
        preferred_element_type=jnp.float32)
    t = e2 - (s + s)
    lanes = jax.lax.broadcasted_iota(jnp.int32, t.shape, 1)
    ids = []
    for _ in range(C):
        m = jnp.min(t, axis=1)
        idc = jnp.min(jnp.where(t == m[:, None], lanes, K), axis=1)
        ids.append(idc)
        t = jnp.where(lanes == idc[:, None], jnp.inf, t)
    cid_ref[0] = jnp.stack(ids, axis=1)


def _top_candidates(x2d, et):
    return pl.pallas_call(
        _topc_kernel,
        grid=(NBR1,),
        in_specs=[
            pl.BlockSpec((RB1, L), lambda r: (r, 0)),
            pl.BlockSpec((L, K), lambda r: (0, 0)),
        ],
        out_specs=pl.BlockSpec((1, RB1, C), lambda r: (r, 0, 0)),
        out_shape=jax.ShapeDtypeStruct((NBR1, RB1, C), jnp.int32),
    )(x2d, et)


_NC, _NS = 2, 16
_NW = _NC * _NS
_M = B * C
_PER_W = _M // _NW
_CHUNK = 216
_NCHUNK = _PER_W // _CHUNK


def _sc_gather(e2d, cid_flat):
    mesh = plsc.VectorSubcoreMesh(core_axis_name="c", subcore_axis_name="s")

    @functools.partial(
        pl.kernel, mesh=mesh,
        out_type=jax.ShapeDtypeStruct((_M, L), jnp.float32),
        scratch_types=[
            pltpu.VMEM((_CHUNK,), jnp.int32),
            pltpu.VMEM((_CHUNK, L), jnp.float32),
            pltpu.SemaphoreType.DMA,
        ],
    )
    def k2(table_hbm, idx_hbm, out_hbm, idx_v, rows_v, sem):
        wid = lax.axis_index("s") * _NC + lax.axis_index("c")
        for t in range(_NCHUNK):
            base = wid * _PER_W + t * _CHUNK
            pltpu.sync_copy(idx_hbm.at[pl.ds(base, _CHUNK)], idx_v)
            pltpu.async_copy(table_hbm.at[idx_v], rows_v, sem).wait()
            pltpu.sync_copy(rows_v, out_hbm.at[pl.ds(base, _CHUNK)])

    return k2(e2d, cid_flat)


def _select_kernel(x_ref, cnd_ref, cid_ref, out0_ref, d2_ref, hist_ref,
                   ent_ref):
    r = pl.program_id(0)
    xb = x_ref[...]
    cnd = cnd_ref[...]
    cid = cid_ref[...]
    diff = xb[:, None, :] - cnd
    d2c = _tree_sum(diff * diff)
    dc = jnp.sqrt(d2c)
    m = jnp.min(dc, axis=1)
    tied = dc == m[:, None]
    jsel = jnp.min(jnp.where(tied, cid, K), axis=1)
    csel = cid == jsel[:, None]
    d2sel = jnp.min(jnp.where(csel, d2c, jnp.inf), axis=1)
    sel = cnd[:, 0, :]
    for c in range(1, C):
        sel = jnp.where(csel[:, c][:, None], cnd[:, c, :], sel)
    out0_ref[...] = (sel - xb) + xb
    d2_ref[...] = d2sel[None, None, :]
    lanes = jax.lax.broadcasted_iota(jnp.int32, (RB3, K), 1)
    oh = jnp.where(jsel[:, None] == lanes, 1.0, 0.0)
    partial = jnp.sum(oh, axis=0)[None, :]

    @pl.when(r == 0)
    def _():
        hist_ref[...] = partial

    @pl.when(r > 0)
    def _():
        hist_ref[...] = hist_ref[...] + partial

    @pl.when(r == NBR3 - 1)
    def _():
        h = hist_ref[...]
        prob = h / B
        safe = jnp.where(h > 0, prob, 1.0)
        ent_ref[0, 0] = -jnp.sum(jnp.where(h > 0, prob * jnp.log(safe), 0.0))


def _select_outputs(x2d, cand, cid2d):
    return pl.pallas_call(
        _select_kernel,
        grid=(NBR3,),
        in_specs=[
            pl.BlockSpec((RB3, L), lambda r: (r, 0)),
            pl.BlockSpec((RB3, C, L), lambda r: (r, 0, 0)),
            pl.BlockSpec((RB3, C), lambda r: (r, 0)),
        ],
        out_specs=[
            pl.BlockSpec((RB3, L), lambda r: (r, 0)),
            pl.BlockSpec((1, 1, RB3), lambda r: (r, 0, 0)),
            pl.BlockSpec((1, K), lambda r: (0, 0)),
            pl.BlockSpec(memory_space=pltpu.SMEM),
        ],
        out_shape=[
            jax.ShapeDtypeStruct((B, L), jnp.float32),
            jax.ShapeDtypeStruct((NBR3, 1, RB3), jnp.float32),
            jax.ShapeDtypeStruct((1, K), jnp.float32),
            jax.ShapeDtypeStruct((1, 1), jnp.float32),
        ],
        compiler_params=pltpu.CompilerParams(
            dimension_semantics=("arbitrary",),
        ),
    )(x2d, cand, cid2d)


def kernel(x0, embedding0):
    x2d = x0.reshape(B, L)
    e2d = embedding0.reshape(K, L)
    cid = _top_candidates(x2d, e2d.T).reshape(B, C)
    cand = _sc_gather(e2d, cid.reshape(_M)).reshape(B, C, L)
    out0, d2min, _hist, ent = _select_outputs(x2d, cand, cid)
    out1 = d2min.reshape(x0.shape[0], x0.shape[1], x0.shape[2])
    return (out0.reshape(x0.shape), out1, out1, ent[0, 0])


# speedup vs baseline: 7.4201x; 2.5004x over previous
import functools

import jax
import jax.numpy as jnp
from jax import lax
from jax.experimental import pallas as pl
from jax.experimental.pallas import tpu as pltpu
from jax.experimental.pallas import tpu_sc as plsc

B = 4608
L = 256
K = 1024
C = 6

RB1 = 512
NBR1 = B // RB1

CB = 512
NB3 = B // CB


def _topc_kernel(x_ref, et_ref, cid_ref, e2_ref):
    r = pl.program_id(0)

    @pl.when(r == 0)
    def _():
        et0 = et_ref[...]
        e2_ref[...] = jnp.sum(et0 * et0, axis=0, keepdims=True)

    xb = x_ref[...]
    s = jax.lax.dot_general(
        xb, et_ref[...], (((1,), (0,)), ((), ())),
        precision=jax.lax.Precision.HIGHEST,
        preferred_element_type=jnp.float32)
    t = e2_ref[...] - (s + s)
    lanes = jax.lax.broadcasted_iota(jnp.int32, t.shape, 1)
    ids = []
    for _ in range(C):
        idc = jnp.argmin(t, axis=1).astype(jnp.int32)
        ids.append(idc)
        t = jnp.where(lanes == idc[:, None], jnp.inf, t)
    cid_ref[0] = jnp.stack(ids, axis=1)


def _top_candidates(x2d, et):
    return pl.pallas_call(
        _topc_kernel,
        grid=(NBR1,),
        in_specs=[
            pl.BlockSpec((RB1, L), lambda r: (r, 0)),
            pl.BlockSpec((L, K), lambda r: (0, 0)),
        ],
        out_specs=pl.BlockSpec((1, RB1, C), lambda r: (r, 0, 0)),
        out_shape=jax.ShapeDtypeStruct((NBR1, RB1, C), jnp.int32),
        scratch_shapes=[pltpu.VMEM((1, K), jnp.float32)],
        compiler_params=pltpu.CompilerParams(
            dimension_semantics=("arbitrary",),
        ),
    )(x2d, et)


_NC, _NS = 2, 16
_NW = _NC * _NS
_M = B * C
_PER_W = _M // _NW
_CHUNK = 216
_NCHUNK = _PER_W // _CHUNK


def _sc_gather(e2d, cid_flat):
    mesh = plsc.VectorSubcoreMesh(core_axis_name="c", subcore_axis_name="s")

    @functools.partial(
        pl.kernel, mesh=mesh,
        out_type=jax.ShapeDtypeStruct((_M, L), jnp.float32),
        scratch_types=[
            pltpu.VMEM((_CHUNK,), jnp.int32),
            pltpu.VMEM((_CHUNK, L), jnp.float32),
            pltpu.SemaphoreType.DMA,
        ],
    )
    def k2(table_hbm, idx_hbm, out_hbm, idx_v, rows_v, sem):
        wid = lax.axis_index("s") * _NC + lax.axis_index("c")
        for t in range(_NCHUNK):
            base = wid * _PER_W + t * _CHUNK
            pltpu.sync_copy(idx_hbm.at[pl.ds(base, _CHUNK)], idx_v)
            pltpu.async_copy(table_hbm.at[idx_v], rows_v, sem).wait()
            pltpu.sync_copy(rows_v, out_hbm.at[pl.ds(base, _CHUNK)])

    return k2(e2d, cid_flat)


def _select_kernel(xt_ref, cnd_ref, cid_ref, out0_ref, d2_ref, hist_ref,
                   ent_ref):
    r = pl.program_id(0)
    xt = xt_ref[...]
    cnd = cnd_ref[...]
    cid = cid_ref[...]
    diff = xt[None, :, :] - cnd
    y = diff * diff
    z = y[:, :128, :] + y[:, 128:, :]
    zr = z.reshape(C, 16, 8, CB)
    acc = zr[:, 0]
    for g in range(1, 16):
        acc = acc + zr[:, g]
    t4 = acc[:, :4, :] + acc[:, 4:, :]
    t2 = t4[:, :2, :] + t4[:, 2:, :]
    d2c = t2[:, 0, :] + t2[:, 1, :]
    dc = jnp.sqrt(d2c)
    m = jnp.min(dc, axis=0)
    tied = dc == m[None, :]
    jsel = jnp.min(jnp.where(tied, cid, K), axis=0)
    csel = cid == jsel[None, :]
    d2sel = jnp.min(jnp.where(csel, d2c, jnp.inf), axis=0)
    sel = cnd[0]
    for c in range(1, C):
        sel = jnp.where(csel[c][None, :], cnd[c], sel)
    out0_ref[...] = (sel - xt) + xt
    d2_ref[...] = d2sel[None, None, :]
    bid = (jax.lax.broadcasted_iota(jnp.int32, (8, 128, 1), 0) * 128
           + jax.lax.broadcasted_iota(jnp.int32, (8, 128, 1), 1))
    oh = jnp.where(bid == jsel[None, None, :], 1.0, 0.0)
    partial = jnp.sum(oh, axis=2)

    @pl.when(r == 0)
    def _():
        hist_ref[...] = partial

    @pl.when(r > 0)
    def _():
        hist_ref[...] = hist_ref[...] + partial

    @pl.when(r == NB3 - 1)
    def _():
        h = hist_ref[...]
        prob = h / B
        safe = jnp.where(h > 0, prob, 1.0)
        ent_ref[0, 0] = -jnp.sum(jnp.where(h > 0, prob * jnp.log(safe), 0.0))


def _select_outputs(x2dt, candt, cidt):
    return pl.pallas_call(
        _select_kernel,
        grid=(NB3,),
        in_specs=[
            pl.BlockSpec((L, CB), lambda r: (0, r)),
            pl.BlockSpec((C, L, CB), lambda r: (0, 0, r)),
            pl.BlockSpec((C, CB), lambda r: (0, r)),
        ],
        out_specs=[
            pl.BlockSpec((L, CB), lambda r: (0, r)),
            pl.BlockSpec((1, 1, CB), lambda r: (r, 0, 0)),
            pl.BlockSpec((8, 128), lambda r: (0, 0)),
            pl.BlockSpec(memory_space=pltpu.SMEM),
        ],
        out_shape=[
            jax.ShapeDtypeStruct((L, B), jnp.float32),
            jax.ShapeDtypeStruct((NB3, 1, CB), jnp.float32),
            jax.ShapeDtypeStruct((8, 128), jnp.float32),
            jax.ShapeDtypeStruct((1, 1), jnp.float32),
        ],
        compiler_params=pltpu.CompilerParams(
            dimension_semantics=("arbitrary",),
        ),
    )(x2dt, candt, cidt)


def kernel(x0, embedding0):
    x2d = x0.reshape(B, L)
    e2d = embedding0.reshape(K, L)
    cid = _top_candidates(x2d, e2d.T).reshape(B, C)
    cand = _sc_gather(e2d, cid.reshape(_M))
    candt = cand.reshape(B, C, L).transpose(1, 2, 0)
    out0t, d2min, _hist, ent = _select_outputs(x2d.T, candt, cid.T)
    out1 = d2min.reshape(x0.shape[0], x0.shape[1], x0.shape[2])
    return (out0t.T.reshape(x0.shape), out1, out1, ent[0, 0])


# speedup vs baseline: 8.0528x; 1.0853x over previous
import functools

import jax
import jax.numpy as jnp
from jax import lax
from jax.experimental import pallas as pl
from jax.experimental.pallas import tpu as pltpu
from jax.experimental.pallas import tpu_sc as plsc

B = 4608
L = 256
K = 1024
C = 6

RB1 = 512
NBR1 = B // RB1

CB = 512
NB3 = B // CB


def _topc_kernel(x_ref, et_ref, cid_ref, e2_ref):
    r = pl.program_id(0)

    @pl.when(r == 0)
    def _():
        et0 = et_ref[...]
        e2_ref[...] = jnp.sum(et0 * et0, axis=0, keepdims=True)

    xb = x_ref[...]
    s = jax.lax.dot_general(
        xb, et_ref[...], (((1,), (0,)), ((), ())),
        precision=jax.lax.Precision.HIGHEST,
        preferred_element_type=jnp.float32)
    t = e2_ref[...] - (s + s)
    lanes = jax.lax.broadcasted_iota(jnp.int32, t.shape, 1)
    ids = []
    for _ in range(C):
        idc = jnp.argmin(t, axis=1).astype(jnp.int32)
        ids.append(idc)
        t = jnp.where(lanes == idc[:, None], jnp.inf, t)
    cid_ref[0] = jnp.stack(ids, axis=1)


def _top_candidates(x2d, et):
    return pl.pallas_call(
        _topc_kernel,
        grid=(NBR1,),
        in_specs=[
            pl.BlockSpec((RB1, L), lambda r: (r, 0)),
            pl.BlockSpec((L, K), lambda r: (0, 0)),
        ],
        out_specs=pl.BlockSpec((1, RB1, C), lambda r: (r, 0, 0)),
        out_shape=jax.ShapeDtypeStruct((NBR1, RB1, C), jnp.int32),
        scratch_shapes=[pltpu.VMEM((1, K), jnp.float32)],
        compiler_params=pltpu.CompilerParams(
            dimension_semantics=("arbitrary",),
        ),
    )(x2d, et)


_NC, _NS = 2, 16
_NW = _NC * _NS
_M = B * C
_PER_W = _M // _NW
_CHUNK = 216
_NCHUNK = _PER_W // _CHUNK


def _sc_gather(e2d, cid_flat):
    mesh = plsc.VectorSubcoreMesh(core_axis_name="c", subcore_axis_name="s")

    @functools.partial(
        pl.kernel, mesh=mesh,
        out_type=jax.ShapeDtypeStruct((_M, L), jnp.float32),
        scratch_types=[
            pltpu.VMEM((_CHUNK,), jnp.int32),
            pltpu.VMEM((_CHUNK, L), jnp.float32),
            pltpu.SemaphoreType.DMA,
        ],
    )
    def k2(table_hbm, idx_hbm, out_hbm, idx_v, rows_v, sem):
        wid = lax.axis_index("s") * _NC + lax.axis_index("c")
        for t in range(_NCHUNK):
            base = wid * _PER_W + t * _CHUNK
            pltpu.sync_copy(idx_hbm.at[pl.ds(base, _CHUNK)], idx_v)
            pltpu.async_copy(table_hbm.at[idx_v], rows_v, sem).wait()
            pltpu.sync_copy(rows_v, out_hbm.at[pl.ds(base, _CHUNK)])

    return k2(e2d, cid_flat)


def _select_kernel(x_ref, cnd_ref, cid_ref, out0_ref, d2_ref, hist_ref,
                   ent_ref):
    r = pl.program_id(0)
    xb = x_ref[...]
    xt = xb.T
    cnd = cnd_ref[...]
    cid = cid_ref[...]
    diff = xt[None, :, :] - cnd
    y = diff * diff
    z = y[:, :128, :] + y[:, 128:, :]
    zr = z.reshape(C, 16, 8, CB)
    acc = zr[:, 0]
    for g in range(1, 16):
        acc = acc + zr[:, g]
    t4 = acc[:, :4, :] + acc[:, 4:, :]
    t2 = t4[:, :2, :] + t4[:, 2:, :]
    d2c = t2[:, 0, :] + t2[:, 1, :]
    dc = jnp.sqrt(d2c)
    m = jnp.min(dc, axis=0)
    tied = dc == m[None, :]
    jsel = jnp.min(jnp.where(tied, cid, K), axis=0)
    csel = cid == jsel[None, :]
    d2sel = jnp.min(jnp.where(csel, d2c, jnp.inf), axis=0)
    sel = cnd[0]
    for c in range(1, C):
        sel = jnp.where(csel[c][None, :], cnd[c], sel)
    out0_ref[...] = (sel.T - xb) + xb
    d2_ref[...] = d2sel[None, None, :]
    bid = (jax.lax.broadcasted_iota(jnp.int32, (8, 128, 1), 0) * 128
           + jax.lax.broadcasted_iota(jnp.int32, (8, 128, 1), 1))
    oh = jnp.where(bid == jsel[None, None, :], 1.0, 0.0)
    partial = jnp.sum(oh, axis=2)

    @pl.when(r == 0)
    def _():
        hist_ref[...] = partial

    @pl.when(r > 0)
    def _():
        hist_ref[...] = hist_ref[...] + partial

    @pl.when(r == NB3 - 1)
    def _():
        h = hist_ref[...]
        prob = h / B
        safe = jnp.where(h > 0, prob, 1.0)
        ent_ref[0, 0] = -jnp.sum(jnp.where(h > 0, prob * jnp.log(safe), 0.0))


def _select_outputs(x2d, candt, cidt):
    return pl.pallas_call(
        _select_kernel,
        grid=(NB3,),
        in_specs=[
            pl.BlockSpec((CB, L), lambda r: (r, 0)),
            pl.BlockSpec((C, L, CB), lambda r: (0, 0, r)),
            pl.BlockSpec((C, CB), lambda r: (0, r)),
        ],
        out_specs=[
            pl.BlockSpec((CB, L), lambda r: (r, 0)),
            pl.BlockSpec((1, 1, CB), lambda r: (r, 0, 0)),
            pl.BlockSpec((8, 128), lambda r: (0, 0)),
            pl.BlockSpec(memory_space=pltpu.SMEM),
        ],
        out_shape=[
            jax.ShapeDtypeStruct((B, L), jnp.float32),
            jax.ShapeDtypeStruct((NB3, 1, CB), jnp.float32),
            jax.ShapeDtypeStruct((8, 128), jnp.float32),
            jax.ShapeDtypeStruct((1, 1), jnp.float32),
        ],
        compiler_params=pltpu.CompilerParams(
            dimension_semantics=("arbitrary",),
        ),
    )(x2d, candt, cidt)


def kernel(x0, embedding0):
    x2d = x0.reshape(B, L)
    e2d = embedding0.reshape(K, L)
    cid = _top_candidates(x2d, e2d.T).reshape(B, C)
    cand = _sc_gather(e2d, cid.reshape(_M))
    candt = cand.reshape(B, C, L).transpose(1, 2, 0)
    out0, d2min, _hist, ent = _select_outputs(x2d, candt, cid.T)
    out1 = d2min.reshape(x0.shape[0], x0.shape[1], x0.shape[2])
    return (out0.reshape(x0.shape), out1, out1, ent[0, 0])


# speedup vs baseline: 8.8198x; 1.0953x over previous
import functools

import jax
import jax.numpy as jnp
from jax import lax
from jax.experimental import pallas as pl
from jax.experimental.pallas import tpu as pltpu
from jax.experimental.pallas import tpu_sc as plsc

B = 4608
L = 256
K = 1024
C = 6

RB1 = 512
NBR1 = B // RB1

CB = 512
NB3 = B // CB


def _topc_kernel(x_ref, e_ref, cid_ref, e2_ref):
    r = pl.program_id(0)

    @pl.when(r == 0)
    def _():
        e0 = e_ref[...]
        e2_ref[...] = jnp.sum(e0 * e0, axis=1, keepdims=True).T

    xb = x_ref[...]
    s = jax.lax.dot_general(
        xb, e_ref[...], (((1,), (1,)), ((), ())),
        precision=jax.lax.Precision.HIGHEST,
        preferred_element_type=jnp.float32)
    t = e2_ref[...] - (s + s)
    lanes = jax.lax.broadcasted_iota(jnp.int32, t.shape, 1)
    ids = []
    for _ in range(C):
        idc = jnp.argmin(t, axis=1).astype(jnp.int32)
        ids.append(idc)
        t = jnp.where(lanes == idc[:, None], jnp.inf, t)
    cid_ref[0] = jnp.stack(ids, axis=1)


def _top_candidates(x2d, e2d):
    return pl.pallas_call(
        _topc_kernel,
        grid=(NBR1,),
        in_specs=[
            pl.BlockSpec((RB1, L), lambda r: (r, 0)),
            pl.BlockSpec((K, L), lambda r: (0, 0)),
        ],
        out_specs=pl.BlockSpec((1, RB1, C), lambda r: (r, 0, 0)),
        out_shape=jax.ShapeDtypeStruct((NBR1, RB1, C), jnp.int32),
        scratch_shapes=[pltpu.VMEM((1, K), jnp.float32)],
        compiler_params=pltpu.CompilerParams(
            dimension_semantics=("arbitrary",),
        ),
    )(x2d, e2d)


_NC, _NS = 2, 16
_NW = _NC * _NS
_M = B * C
_PER_W = _M // _NW
_CHUNK = 216
_NCHUNK = _PER_W // _CHUNK


def _sc_gather(e2d, cid_flat):
    mesh = plsc.VectorSubcoreMesh(core_axis_name="c", subcore_axis_name="s")

    @functools.partial(
        pl.kernel, mesh=mesh,
        out_type=jax.ShapeDtypeStruct((_M, L), jnp.float32),
        scratch_types=[
            pltpu.VMEM((_CHUNK,), jnp.int32),
            pltpu.VMEM((_CHUNK, L), jnp.float32),
            pltpu.SemaphoreType.DMA,
        ],
    )
    def k2(table_hbm, idx_hbm, out_hbm, idx_v, rows_v, sem):
        wid = lax.axis_index("s") * _NC + lax.axis_index("c")
        for t in range(_NCHUNK):
            base = wid * _PER_W + t * _CHUNK
            pltpu.sync_copy(idx_hbm.at[pl.ds(base, _CHUNK)], idx_v)
            pltpu.async_copy(table_hbm.at[idx_v], rows_v, sem).wait()
            pltpu.sync_copy(rows_v, out_hbm.at[pl.ds(base, _CHUNK)])

    return k2(e2d, cid_flat)


def _select_kernel(x_ref, cnd_ref, cid_ref, out0_ref, d2_ref, hist_ref,
                   ent_ref):
    r = pl.program_id(0)
    xb = x_ref[...]
    xt = xb.T
    cid = cid_ref[...].T
    planes = []
    d2s = []
    for c in range(C):
        pc = cnd_ref[:, c, :].T
        planes.append(pc)
        diffc = xt - pc
        yc = diffc * diffc
        zc = yc[:128, :] + yc[128:, :]
        zr = zc.reshape(16, 8, CB)
        acc = zr[0]
        for g in range(1, 16):
            acc = acc + zr[g]
        t4 = acc[:4, :] + acc[4:, :]
        t2 = t4[:2, :] + t4[2:, :]
        d2s.append(t2[0, :] + t2[1, :])
    d2c = jnp.stack(d2s, axis=0)
    dc = jnp.sqrt(d2c)
    m = jnp.min(dc, axis=0)
    tied = dc == m[None, :]
    jsel = jnp.min(jnp.where(tied, cid, K), axis=0)
    csel = cid == jsel[None, :]
    d2sel = jnp.min(jnp.where(csel, d2c, jnp.inf), axis=0)
    sel = planes[0]
    for c in range(1, C):
        sel = jnp.where(csel[c][None, :], planes[c], sel)
    out0_ref[...] = (sel.T - xb) + xb
    d2_ref[...] = d2sel[None, None, :]
    bid = (jax.lax.broadcasted_iota(jnp.int32, (8, 128, 1), 0) * 128
           + jax.lax.broadcasted_iota(jnp.int32, (8, 128, 1), 1))
    oh = jnp.where(bid == jsel[None, None, :], 1.0, 0.0)
    partial = jnp.sum(oh, axis=2)

    @pl.when(r == 0)
    def _():
        hist_ref[...] = partial

    @pl.when(r > 0)
    def _():
        hist_ref[...] = hist_ref[...] + partial

    @pl.when(r == NB3 - 1)
    def _():
        h = hist_ref[...]
        prob = h / B
        safe = jnp.where(h > 0, prob, 1.0)
        ent_ref[0, 0] = -jnp.sum(jnp.where(h > 0, prob * jnp.log(safe), 0.0))


def _select_outputs(x2d, cand, cid):
    return pl.pallas_call(
        _select_kernel,
        grid=(NB3,),
        in_specs=[
            pl.BlockSpec((CB, L), lambda r: (r, 0)),
            pl.BlockSpec((CB, C, L), lambda r: (r, 0, 0)),
            pl.BlockSpec((CB, C), lambda r: (r, 0)),
        ],
        out_specs=[
            pl.BlockSpec((CB, L), lambda r: (r, 0)),
            pl.BlockSpec((1, 1, CB), lambda r: (r, 0, 0)),
            pl.BlockSpec((8, 128), lambda r: (0, 0)),
            pl.BlockSpec(memory_space=pltpu.SMEM),
        ],
        out_shape=[
            jax.ShapeDtypeStruct((B, L), jnp.float32),
            jax.ShapeDtypeStruct((NB3, 1, CB), jnp.float32),
            jax.ShapeDtypeStruct((8, 128), jnp.float32),
            jax.ShapeDtypeStruct((1, 1), jnp.float32),
        ],
        compiler_params=pltpu.CompilerParams(
            dimension_semantics=("arbitrary",),
        ),
    )(x2d, cand, cid)


def kernel(x0, embedding0):
    x2d = x0.reshape(B, L)
    e2d = embedding0.reshape(K, L)
    cid = _top_candidates(x2d, e2d).reshape(B, C)
    cand = _sc_gather(e2d, cid.reshape(_M))
    out0, d2min, _hist, ent = _select_outputs(x2d, cand.reshape(B, C, L), cid)
    out1 = d2min.reshape(x0.shape[0], x0.shape[1], x0.shape[2])
    return (out0.reshape(x0.shape), out1, out1, ent[0, 0])
